# baseline (speedup 1.0000x reference)
import functools

import jax
import jax.numpy as jnp
from jax import lax
from jax.experimental import pallas as pl
from jax.experimental.pallas import tpu as pltpu
from jax.experimental.pallas import tpu_sc as plsc

N = 10000
NPAD = 10240
DUMP = 10016
FIN = 128
HID = 128
FOUT = 64
E = 320000
NW = 32
CHUNK = 100
NCH = 100
NHALF = 2
EPW = NCH * CHUNK
IOBLK = 80
DEGROWS = NPAD // 128

_mesh = plsc.VectorSubcoreMesh(core_axis_name="c", subcore_axis_name="s")
_sc_params = pltpu.CompilerParams(needs_layout_passes=False,
                                  use_tc_tiling_on_sc=False)
_sc_params_tc = pltpu.CompilerParams(needs_layout_passes=False,
                                     use_tc_tiling_on_sc=True)


@functools.partial(
    pl.kernel,
    mesh=_mesh,
    compiler_params=_sc_params,
    out_type=jax.ShapeDtypeStruct((2, DEGROWS, 128), jnp.float32),
    scratch_types=[
        pltpu.VMEM((E // NW,), jnp.int32),
        pltpu.VMEM((DEGROWS, 128), jnp.float32),
        pltpu.VMEM((DEGROWS,), jnp.int32),
        pltpu.VMEM((8, 128), jnp.float32),
        pltpu.VMEM_SHARED((DEGROWS, 128), jnp.float32),
    ],
)
def _deg_kernel(ei_hbm, deg_out, dstbuf, table, iota_r, stage, degacc):
    c = lax.axis_index("c")
    s = lax.axis_index("s")
    wid = c * 16 + s
    pltpu.sync_copy(ei_hbm.at[1, pl.ds(wid * (E // NW), E // NW)], dstbuf)
    zeros = jnp.zeros((16,), jnp.float32)
    for r in range(DEGROWS):
        for j in range(8):
            table[r, 16 * j:16 * (j + 1)] = zeros
    for i in range(DEGROWS // 16):
        iota_r[16 * i:16 * (i + 1)] = lax.iota(jnp.int32, 16) + 16 * i

    @pl.when(s == 0)
    def _():
        pltpu.sync_copy(table, degacc)

    plsc.subcore_barrier()

    ones = jnp.ones((16,), jnp.float32)

    def body(i, carry):
        v = dstbuf[pl.ds(i * 16, 16)]
        hi = lax.shift_right_logical(v, 7)
        lo = lax.bitwise_and(v, 127)
        plsc.addupdate_scatter(table, [hi, lo], ones)
        return carry

    lax.fori_loop(0, E // NW // 16, body, jnp.int32(0))

    pltpu.sync_copy(table, degacc.at[iota_r], add=True)
    plsc.subcore_barrier()

    @pl.when(s < DEGROWS // 8)
    def _():
        pltpu.sync_copy(degacc.at[pl.ds(s * 8, 8)], stage)
        pltpu.sync_copy(stage, deg_out.at[c, pl.ds(s * 8, 8)])


def _make_scatter(F, nbuf):
    @functools.partial(
        pl.kernel,
        mesh=_mesh,
        compiler_params=_sc_params_tc if F % 128 == 0 else _sc_params,
        out_type=jax.ShapeDtypeStruct((2, NPAD, F), jnp.float32),
        scratch_types=[
            pltpu.VMEM((NCH // NHALF, CHUNK), jnp.int32),
            pltpu.VMEM((NCH // NHALF, CHUNK), jnp.int32),
        ] + [pltpu.VMEM((CHUNK, F), jnp.float32) for _ in range(nbuf)]
          + [pltpu.SemaphoreType.DMA for _ in range(nbuf)]
          + [pltpu.VMEM_SHARED((NPAD, F), jnp.float32)],
    )
    def _scatter(hs_hbm, src_hbm, dst_hbm, out_hbm, src_v, dst_v, *rest):
        bufs = rest[:nbuf]
        sems = rest[nbuf:2 * nbuf]
        acc = rest[2 * nbuf]
        c = lax.axis_index("c")
        s = lax.axis_index("s")
        wid = c * 16 + s

        rows_per_tile = NPAD // 16
        base = s * rows_per_tile
        zeros = jnp.zeros((16,), jnp.float32)
        for r in range(IOBLK):
            for j in range(F // 16):
                bufs[0][r, 16 * j:16 * (j + 1)] = zeros
        for k in range(rows_per_tile // IOBLK):
            pltpu.sync_copy(bufs[0].at[pl.ds(0, IOBLK)],
                            acc.at[pl.ds(base + IOBLK * k, IOBLK)])

        plsc.subcore_barrier()

        nh = NCH // NHALF
        for h in range(NHALF):
            pltpu.sync_copy(src_hbm.at[wid, h], src_v)
            pltpu.sync_copy(dst_hbm.at[wid, h], dst_v)
            for b in range(nbuf - 1):
                pltpu.async_copy(hs_hbm.at[src_v.at[b]], bufs[b], sems[b])

            def body(t, carry):
                for b in range(nbuf):
                    j = t * nbuf + b
                    jn = j + nbuf - 1
                    bn = (b + nbuf - 1) % nbuf

                    @pl.when(jn < nh)
                    def _():
                        pltpu.async_copy(hs_hbm.at[src_v.at[jn]], bufs[bn],
                                         sems[bn])

                    pltpu.make_async_copy(hs_hbm.at[src_v.at[j]], bufs[b],
                                          sems[b]).wait()
                    pltpu.sync_copy(bufs[b], acc.at[dst_v.at[j]], add=True)
                return carry

            lax.fori_loop(0, nh // nbuf, body, jnp.int32(0))

        plsc.subcore_barrier()

        for k in range(rows_per_tile // IOBLK):
            b = bufs[k % nbuf].at[pl.ds(0, IOBLK)]
            pltpu.sync_copy(acc.at[pl.ds(base + IOBLK * k, IOBLK)], b)
            pltpu.sync_copy(b, out_hbm.at[c, pl.ds(base + IOBLK * k, IOBLK)])

    return _scatter


_scatter_hid = _make_scatter(HID, 2)
_scatter_out = _make_scatter(FOUT, 5)


_BS = 1000


def _mm1_body(x_ref, d0_ref, d1_ref, w_ref, hs_ref, dinv_ref):
    dinv = lax.rsqrt(d0_ref[...] + d1_ref[...] + 1.0)
    h = jnp.dot(x_ref[...], w_ref[...],
                preferred_element_type=jnp.float32,
                precision=lax.Precision.HIGHEST)
    hs_ref[...] = h * dinv
    dinv_ref[...] = dinv


def _mm2_body(a_ref, hs_ref, dinv_ref, b_ref, w_ref, out_ref):
    dinv = dinv_ref[...]
    z = dinv * (a_ref[0] + a_ref[1] + hs_ref[...]) + b_ref[...]
    z = jnp.maximum(z, 0.0)
    h2 = jnp.dot(z, w_ref[...],
                 preferred_element_type=jnp.float32,
                 precision=lax.Precision.HIGHEST)
    out_ref[...] = h2 * dinv


def _fin_body(a_ref, hs_ref, dinv_ref, b_ref, out_ref):
    out_ref[...] = (dinv_ref[...] * (a_ref[0] + a_ref[1] + hs_ref[...])
                    + b_ref[...])


def _row_spec(width):
    return pl.BlockSpec((_BS, width), lambda b: (b, 0))


def _acc_spec(width):
    return pl.BlockSpec((2, _BS, width), lambda b: (0, b, 0))


def _full_spec(shape):
    return pl.BlockSpec(shape, lambda b: (0,) * len(shape))


_mm1 = pl.pallas_call(
    _mm1_body,
    grid=(N // _BS,),
    in_specs=[_row_spec(FIN), _row_spec(1), _row_spec(1),
              _full_spec((FIN, HID))],
    out_specs=[_row_spec(HID), _row_spec(1)],
    out_shape=[jax.ShapeDtypeStruct((N, HID), jnp.float32),
               jax.ShapeDtypeStruct((N, 1), jnp.float32)],
)

_mm2 = pl.pallas_call(
    _mm2_body,
    grid=(N // _BS,),
    in_specs=[_acc_spec(HID), _row_spec(HID), _row_spec(1),
              _full_spec((1, HID)), _full_spec((HID, FOUT))],
    out_specs=_row_spec(FOUT),
    out_shape=jax.ShapeDtypeStruct((N, FOUT), jnp.float32),
)

_fin = pl.pallas_call(
    _fin_body,
    grid=(N // _BS,),
    in_specs=[_acc_spec(FOUT), _row_spec(FOUT), _row_spec(1),
              _full_spec((1, FOUT))],
    out_specs=_row_spec(FOUT),
    out_shape=jax.ShapeDtypeStruct((N, FOUT), jnp.float32),
)


def kernel(x, edge_index, W1, b1, W2, b2):
    src3 = edge_index[0].reshape(NW, NHALF, NCH // NHALF, CHUNK)
    dst3 = edge_index[1].reshape(NW, NHALF, NCH // NHALF, CHUNK)

    deg2 = _deg_kernel(edge_index)
    d0 = deg2[0].reshape(NPAD, 1)
    d1 = deg2[1].reshape(NPAD, 1)

    hs1, dinv = _mm1(x, d0, d1, W1)
    acc1 = _scatter_hid(hs1, src3, dst3)
    hs2 = _mm2(acc1, hs1, dinv, b1.reshape(1, HID), W2)
    acc2 = _scatter_out(hs2, src3, dst3)
    return _fin(acc2, hs2, dinv, b2.reshape(1, FOUT))


# speedup vs baseline: 1.0188x; 1.0188x over previous
import functools

import jax
import jax.numpy as jnp
from jax import lax
from jax.experimental import pallas as pl
from jax.experimental.pallas import tpu as pltpu
from jax.experimental.pallas import tpu_sc as plsc

N = 10000
NPAD = 10240
DUMP = 10016
FIN = 128
HID = 128
FOUT = 64
E = 320000
NW = 32
CHUNK = 128
NCH = 80
NHALF = 2
EPW = NCH * CHUNK
EPAD = NW * EPW
IOBLK = 80
DEGROWS = NPAD // 128

_mesh = plsc.VectorSubcoreMesh(core_axis_name="c", subcore_axis_name="s")
_sc_params = pltpu.CompilerParams(needs_layout_passes=False,
                                  use_tc_tiling_on_sc=False)
_sc_params_tc = pltpu.CompilerParams(needs_layout_passes=False,
                                     use_tc_tiling_on_sc=True)


@functools.partial(
    pl.kernel,
    mesh=_mesh,
    compiler_params=_sc_params,
    out_type=jax.ShapeDtypeStruct((2, DEGROWS, 128), jnp.float32),
    scratch_types=[
        pltpu.VMEM((E // NW,), jnp.int32),
        pltpu.VMEM((DEGROWS, 128), jnp.float32),
        pltpu.VMEM((DEGROWS,), jnp.int32),
        pltpu.VMEM((8, 128), jnp.float32),
        pltpu.VMEM_SHARED((DEGROWS, 128), jnp.float32),
    ],
)
def _deg_kernel(ei_hbm, deg_out, dstbuf, table, iota_r, stage, degacc):
    c = lax.axis_index("c")
    s = lax.axis_index("s")
    wid = c * 16 + s
    pltpu.sync_copy(ei_hbm.at[1, pl.ds(wid * (E // NW), E // NW)], dstbuf)
    zeros = jnp.zeros((16,), jnp.float32)
    for r in range(DEGROWS):
        for j in range(8):
            table[r, 16 * j:16 * (j + 1)] = zeros
    for i in range(DEGROWS // 16):
        iota_r[16 * i:16 * (i + 1)] = lax.iota(jnp.int32, 16) + 16 * i

    @pl.when(s == 0)
    def _():
        pltpu.sync_copy(table, degacc)

    plsc.subcore_barrier()

    ones = jnp.ones((16,), jnp.float32)

    def body(i, carry):
        v = dstbuf[pl.ds(i * 16, 16)]
        hi = lax.shift_right_logical(v, 7)
        lo = lax.bitwise_and(v, 127)
        plsc.addupdate_scatter(table, [hi, lo], ones)
        return carry

    lax.fori_loop(0, E // NW // 16, body, jnp.int32(0))

    pltpu.sync_copy(table, degacc.at[iota_r], add=True)
    plsc.subcore_barrier()

    @pl.when(s < DEGROWS // 8)
    def _():
        pltpu.sync_copy(degacc.at[pl.ds(s * 8, 8)], stage)
        pltpu.sync_copy(stage, deg_out.at[c, pl.ds(s * 8, 8)])


def _make_scatter(F, nbuf):
    @functools.partial(
        pl.kernel,
        mesh=_mesh,
        compiler_params=_sc_params_tc if F % 128 == 0 else _sc_params,
        out_type=jax.ShapeDtypeStruct((2, NPAD, F), jnp.float32),
        scratch_types=[
            pltpu.VMEM((NCH // NHALF, CHUNK), jnp.int32),
            pltpu.VMEM((NCH // NHALF, CHUNK), jnp.int32),
        ] + [pltpu.VMEM((CHUNK, F), jnp.float32) for _ in range(nbuf)]
          + [pltpu.SemaphoreType.DMA for _ in range(nbuf)]
          + [pltpu.VMEM_SHARED((NPAD, F), jnp.float32)],
    )
    def _scatter(hs_hbm, src_hbm, dst_hbm, out_hbm, src_v, dst_v, *rest):
        bufs = rest[:nbuf]
        sems = rest[nbuf:2 * nbuf]
        acc = rest[2 * nbuf]
        c = lax.axis_index("c")
        s = lax.axis_index("s")
        wid = c * 16 + s

        rows_per_tile = NPAD // 16
        base = s * rows_per_tile
        zeros = jnp.zeros((16,), jnp.float32)
        for r in range(IOBLK):
            for j in range(F // 16):
                bufs[0][r, 16 * j:16 * (j + 1)] = zeros
        for k in range(rows_per_tile // IOBLK):
            pltpu.sync_copy(bufs[0].at[pl.ds(0, IOBLK)],
                            acc.at[pl.ds(base + IOBLK * k, IOBLK)])

        plsc.subcore_barrier()

        nh = NCH // NHALF
        for h in range(NHALF):
            pltpu.sync_copy(src_hbm.at[wid, h], src_v)
            pltpu.sync_copy(dst_hbm.at[wid, h], dst_v)
            for b in range(nbuf - 1):
                pltpu.async_copy(hs_hbm.at[src_v.at[b]], bufs[b], sems[b])

            def body(t, carry):
                for b in range(nbuf):
                    j = t * nbuf + b
                    jn = j + nbuf - 1
                    bn = (b + nbuf - 1) % nbuf

                    @pl.when(jn < nh)
                    def _():
                        pltpu.async_copy(hs_hbm.at[src_v.at[jn]], bufs[bn],
                                         sems[bn])

                    pltpu.make_async_copy(hs_hbm.at[src_v.at[j]], bufs[b],
                                          sems[b]).wait()
                    pltpu.sync_copy(bufs[b], acc.at[dst_v.at[j]], add=True)
                return carry

            lax.fori_loop(0, nh // nbuf, body, jnp.int32(0))

        plsc.subcore_barrier()

        for k in range(rows_per_tile // IOBLK):
            b = bufs[k % nbuf].at[pl.ds(0, IOBLK)]
            pltpu.sync_copy(acc.at[pl.ds(base + IOBLK * k, IOBLK)], b)
            pltpu.sync_copy(b, out_hbm.at[c, pl.ds(base + IOBLK * k, IOBLK)])

    return _scatter


_scatter_hid = _make_scatter(HID, 2)
_scatter_out = _make_scatter(FOUT, 4)


_BS = 1000


def _mm1_body(x_ref, d0_ref, d1_ref, w_ref, hs_ref, dinv_ref):
    dinv = lax.rsqrt(d0_ref[...] + d1_ref[...] + 1.0)
    h = jnp.dot(x_ref[...], w_ref[...],
                preferred_element_type=jnp.float32,
                precision=lax.Precision.HIGHEST)
    hs_ref[...] = h * dinv
    dinv_ref[...] = dinv


def _mm2_body(a_ref, hs_ref, dinv_ref, b_ref, w_ref, out_ref):
    dinv = dinv_ref[...]
    z = dinv * (a_ref[0] + a_ref[1] + hs_ref[...]) + b_ref[...]
    z = jnp.maximum(z, 0.0)
    h2 = jnp.dot(z, w_ref[...],
                 preferred_element_type=jnp.float32,
                 precision=lax.Precision.HIGHEST)
    out_ref[...] = h2 * dinv


def _fin_body(a_ref, hs_ref, dinv_ref, b_ref, out_ref):
    out_ref[...] = (dinv_ref[...] * (a_ref[0] + a_ref[1] + hs_ref[...])
                    + b_ref[...])


def _row_spec(width):
    return pl.BlockSpec((_BS, width), lambda b: (b, 0))


def _acc_spec(width):
    return pl.BlockSpec((2, _BS, width), lambda b: (0, b, 0))


def _full_spec(shape):
    return pl.BlockSpec(shape, lambda b: (0,) * len(shape))


_mm1 = pl.pallas_call(
    _mm1_body,
    grid=(N // _BS,),
    in_specs=[_row_spec(FIN), _row_spec(1), _row_spec(1),
              _full_spec((FIN, HID))],
    out_specs=[_row_spec(HID), _row_spec(1)],
    out_shape=[jax.ShapeDtypeStruct((N, HID), jnp.float32),
               jax.ShapeDtypeStruct((N, 1), jnp.float32)],
)

_mm2 = pl.pallas_call(
    _mm2_body,
    grid=(N // _BS,),
    in_specs=[_acc_spec(HID), _row_spec(HID), _row_spec(1),
              _full_spec((1, HID)), _full_spec((HID, FOUT))],
    out_specs=_row_spec(FOUT),
    out_shape=jax.ShapeDtypeStruct((N, FOUT), jnp.float32),
)

_fin = pl.pallas_call(
    _fin_body,
    grid=(N // _BS,),
    in_specs=[_acc_spec(FOUT), _row_spec(FOUT), _row_spec(1),
              _full_spec((1, FOUT))],
    out_specs=_row_spec(FOUT),
    out_shape=jax.ShapeDtypeStruct((N, FOUT), jnp.float32),
)


def kernel(x, edge_index, W1, b1, W2, b2):
    src = edge_index[0]
    dst = edge_index[1]
    pad = EPAD - E
    src_p = jnp.concatenate([src, jnp.arange(pad, dtype=jnp.int32)])
    dst_p = jnp.concatenate(
        [dst, N + (jnp.arange(pad, dtype=jnp.int32) % (NPAD - N))])
    src3 = src_p.reshape(NW, NHALF, NCH // NHALF, CHUNK)
    dst3 = dst_p.reshape(NW, NHALF, NCH // NHALF, CHUNK)

    deg2 = _deg_kernel(edge_index)
    d0 = deg2[0].reshape(NPAD, 1)
    d1 = deg2[1].reshape(NPAD, 1)

    hs1, dinv = _mm1(x, d0, d1, W1)
    acc1 = _scatter_hid(hs1, src3, dst3)
    hs2 = _mm2(acc1, hs1, dinv, b1.reshape(1, HID), W2)
    acc2 = _scatter_out(hs2, src3, dst3)
    return _fin(acc2, hs2, dinv, b2.reshape(1, FOUT))
